# baseline (speedup 1.0000x reference)
import functools
import math

import jax
import jax.numpy as jnp
from jax import lax
from jax.experimental import pallas as pl
from jax.experimental.pallas import tpu as pltpu
from jax.experimental.pallas import tpu_sc as plsc

N = 2048
H = 768
F = 3072
E = 8
CAP = 256
NSLOT = 2 * N
NW = 32
RPW = NSLOT // NW
_SQRT2 = math.sqrt(2.0)


def _router_body(x_ref, rw_ref, rb_ref, logits_ref, gtop_ref, topk_ref,
                 dest_ref, meta_ref, xbf_ref):
    x = x_ref[...]
    xbf_ref[...] = x.astype(jnp.bfloat16)
    logits = jnp.dot(x, rw_ref[...], preferred_element_type=jnp.float32)
    logits = logits + rb_ref[...]
    logits_ref[...] = logits
    m = jnp.max(logits, axis=-1, keepdims=True)
    ex = jnp.exp(logits - m)
    g = ex / jnp.sum(ex, axis=-1, keepdims=True)

    lane = lax.broadcasted_iota(jnp.int32, (N, E), 1)
    m1 = jnp.max(g, axis=-1, keepdims=True)
    top1 = jnp.min(jnp.where(g == m1, lane, E), axis=-1, keepdims=True)
    oh1 = lane == top1
    g2 = jnp.where(oh1, -jnp.inf, g)
    m2 = jnp.max(g2, axis=-1, keepdims=True)
    top2 = jnp.min(jnp.where(g2 == m2, lane, E), axis=-1, keepdims=True)
    oh2 = lane == top2

    hard = (oh1 | oh2).astype(jnp.float32)
    gt = g * hard
    gt = gt / (jnp.sum(gt, axis=-1, keepdims=True) + 1e-9)
    gtop_ref[...] = gt
    topk_ref[...] = jnp.concatenate([top1, top2], axis=1)

    col = lax.broadcasted_iota(jnp.int32, (CAP, N), 1)

    def cum_excl(cols):
        outs = []
        for i in range(N // CAP):
            row = lax.broadcasted_iota(jnp.int32, (CAP, N), 0) + (i * CAP)
            lb = (col < row).astype(jnp.float32)
            outs.append(jnp.dot(lb, cols, preferred_element_type=jnp.float32))
        return jnp.concatenate(outs, axis=0)

    prim = oh1.astype(jnp.float32)
    counts = jnp.sum(prim, axis=0, keepdims=True)
    used = jnp.minimum(counts, float(CAP))
    free = float(CAP) - used

    cum_p = cum_excl(prim)
    rank_p = jnp.sum(cum_p * prim, axis=-1, keepdims=True)
    keep = rank_p < float(CAP)
    overflow = jnp.logical_not(keep)

    sec = oh2.astype(jnp.float32) * overflow.astype(jnp.float32)
    cum_s = cum_excl(sec)
    rank_s = jnp.sum(cum_s * oh2.astype(jnp.float32), axis=-1, keepdims=True)
    free_sec = jnp.sum(oh2.astype(jnp.float32) * free, axis=-1, keepdims=True)
    used_sec = jnp.sum(oh2.astype(jnp.float32) * used, axis=-1, keepdims=True)
    take2 = overflow & (rank_s < free_sec)

    fb = (overflow & jnp.logical_not(take2)).astype(jnp.float32)
    cum_fb = cum_excl(fb)

    dest = jnp.where(
        keep, top1 * CAP + rank_p.astype(jnp.int32),
        jnp.where(take2,
                  top2 * CAP + (used_sec + rank_s).astype(jnp.int32),
                  N + cum_fb.astype(jnp.int32)))
    dest_ref[...] = dest

    cnt2 = jnp.sum(sec * take2.astype(jnp.float32), axis=0, keepdims=True)
    util = (used + cnt2) / float(N)
    p_mean = jnp.mean(gt, axis=0, keepdims=True)
    fb_cnt = jnp.sum(fb, axis=0, keepdims=True)
    fb_row = jnp.broadcast_to(fb_cnt, (1, E))
    meta_ref[...] = jnp.concatenate(
        [util, p_mean, fb_row, jnp.zeros((5, E), jnp.float32)], axis=0)


def _router_call(x_f, router_w, router_b):
    return pl.pallas_call(
        _router_body,
        out_shape=[
            jax.ShapeDtypeStruct((N, E), jnp.float32),
            jax.ShapeDtypeStruct((N, E), jnp.float32),
            jax.ShapeDtypeStruct((N, 2), jnp.int32),
            jax.ShapeDtypeStruct((N, 1), jnp.int32),
            jax.ShapeDtypeStruct((8, E), jnp.float32),
            jax.ShapeDtypeStruct((N, H), jnp.bfloat16),
        ],
    )(x_f, router_w, router_b)


def _dispatch_body(cnt_hbm, dest_hbm, x_hbm, disp_out, xd_out,
                   cnt_v, dest_v, loc_v, gidx_v, rows_v, sem):
    c = lax.axis_index("c")
    s = lax.axis_index("s")
    wid = s * 2 + c
    base = pl.multiple_of(wid * RPW, RPW)

    pltpu.sync_copy(cnt_hbm, cnt_v)
    fb_cnt = jnp.max(cnt_v[...])
    live = (wid < NW // 2) | (base - N < fb_cnt)

    @pl.when(live)
    def _():
        pltpu.sync_copy(dest_hbm, dest_v)

        def init_body(i, carry):
            loc_v[pl.ds(i * 16, 16)] = jnp.full((16,), N, jnp.int32)
            return carry

        lax.fori_loop(0, (RPW + 16) // 16, init_body, 0)

        lanes = lax.iota(jnp.int32, 16)

        def scat_body(i, carry):
            d = dest_v[pl.ds(i * 16, 16)] - base
            ok = (d >= 0) & (d < RPW)
            d = jnp.where(ok, d, RPW + lanes)
            toks = jnp.where(ok, lanes + i * 16, N)
            plsc.store_scatter(loc_v, [d], toks)
            return carry

        lax.fori_loop(0, N // 16, scat_body, 0)

        pltpu.sync_copy(loc_v.at[pl.ds(0, RPW)], disp_out.at[pl.ds(base, RPW)])

        def san_body(i, carry):
            v = loc_v[pl.ds(i * 16, 16)]
            gidx_v[pl.ds(i * 16, 16)] = jnp.where(v >= N, 0, v)
            return carry

        lax.fori_loop(0, RPW // 16, san_body, 0)

        pltpu.async_copy(x_hbm.at[gidx_v], rows_v, sem).wait()
        pltpu.sync_copy(rows_v, xd_out.at[pl.ds(base, RPW)])


def _dispatch_call(cnt16, dest, x_bf):
    mesh = plsc.VectorSubcoreMesh(core_axis_name="c", subcore_axis_name="s")
    fn = functools.partial(
        pl.kernel,
        mesh=mesh,
        out_type=[
            jax.ShapeDtypeStruct((NSLOT,), jnp.int32),
            jax.ShapeDtypeStruct((NSLOT, H // 2), jnp.int32),
        ],
        scratch_types=[
            pltpu.VMEM((16,), jnp.int32),
            pltpu.VMEM((N,), jnp.int32),
            pltpu.VMEM((RPW + 16,), jnp.int32),
            pltpu.VMEM((RPW,), jnp.int32),
            pltpu.VMEM((RPW, H // 2), jnp.int32),
            pltpu.SemaphoreType.DMA,
        ],
        compiler_params=pltpu.CompilerParams(needs_layout_passes=False),
    )(_dispatch_body)
    return fn(cnt16, dest, x_bf)


def _gelu(h):
    return 0.5 * h * (1.0 + lax.erf(h / _SQRT2))


def _ffn_body(x_ref, w1_ref, b1_ref, w2_ref, b2_ref, o_ref):
    h = jnp.dot(x_ref[...], w1_ref[0].astype(jnp.bfloat16),
                preferred_element_type=jnp.float32)
    h = _gelu(h + b1_ref[0]).astype(jnp.bfloat16)
    o = jnp.dot(h, w2_ref[0].astype(jnp.bfloat16),
                preferred_element_type=jnp.float32)
    o_ref[...] = o + b2_ref[0]


def _ffn_call(x_disp, w1, b1, w2, b2):
    return pl.pallas_call(
        _ffn_body,
        grid=(E,),
        in_specs=[
            pl.BlockSpec((CAP, H), lambda i: (i, 0)),
            pl.BlockSpec((1, H, F), lambda i: (i, 0, 0)),
            pl.BlockSpec((1, 1, F), lambda i: (i, 0, 0)),
            pl.BlockSpec((1, F, H), lambda i: (i, 0, 0)),
            pl.BlockSpec((1, 1, H), lambda i: (i, 0, 0)),
        ],
        out_specs=pl.BlockSpec((CAP, H), lambda i: (i, 0)),
        out_shape=jax.ShapeDtypeStruct((N, H), jnp.float32),
    )(x_disp, w1.reshape(E, H, F), b1.reshape(E, 1, F),
      w2.reshape(E, F, H), b2.reshape(E, 1, H))


def _fb_body(cnt_ref, x_ref, w1_ref, b1_ref, w2_ref, b2_ref, o_ref):
    i = pl.program_id(0)
    live = i * CAP < cnt_ref[0, 0]

    @pl.when(live)
    def _():
        h = jnp.dot(x_ref[...], w1_ref[...].astype(jnp.bfloat16),
                    preferred_element_type=jnp.float32)
        h = _gelu(h + b1_ref[...]).astype(jnp.bfloat16)
        o = jnp.dot(h, w2_ref[...].astype(jnp.bfloat16),
                    preferred_element_type=jnp.float32)
        o_ref[...] = o + b2_ref[...]

    @pl.when(jnp.logical_not(live))
    def _():
        o_ref[...] = jnp.zeros_like(o_ref)


def _fb_call(cnt, x_disp, sw1, sb1, sw2, sb2):
    return pl.pallas_call(
        _fb_body,
        grid=(E,),
        in_specs=[
            pl.BlockSpec(memory_space=pltpu.SMEM),
            pl.BlockSpec((CAP, H), lambda i: (i + E, 0)),
            pl.BlockSpec((H, F), lambda i: (0, 0)),
            pl.BlockSpec((1, F), lambda i: (0, 0)),
            pl.BlockSpec((F, H), lambda i: (0, 0)),
            pl.BlockSpec((1, H), lambda i: (0, 0)),
        ],
        out_specs=pl.BlockSpec((CAP, H), lambda i: (i, 0)),
        out_shape=jax.ShapeDtypeStruct((N, H), jnp.float32),
    )(cnt, x_disp, sw1, sb1, sw2, sb2)


YPAD = N + 8


def _combine_body(cnt_hbm, disp_hbm, oe_hbm, ofb_hbm, y_hbm,
                  cnt_v, idx_v, rows_v, sem):
    c = lax.axis_index("c")
    s = lax.axis_index("s")
    wid = s * 2 + c
    base = pl.multiple_of(wid * RPW, RPW)

    pltpu.sync_copy(cnt_hbm, cnt_v)
    fb_cnt = jnp.max(cnt_v[...])
    live = (wid < NW // 2) | (base - N < fb_cnt)

    @pl.when(live)
    def _():
        pltpu.sync_copy(disp_hbm.at[pl.ds(base, RPW)], idx_v)

        @pl.when(wid < NW // 2)
        def _():
            pltpu.sync_copy(oe_hbm.at[pl.ds(base, RPW)], rows_v)

        @pl.when(wid >= NW // 2)
        def _():
            fb_base = pl.multiple_of(jnp.maximum(base - N, 0), RPW)
            pltpu.sync_copy(ofb_hbm.at[pl.ds(fb_base, RPW)], rows_v)

        pltpu.async_copy(rows_v, y_hbm.at[idx_v], sem).wait()


def _combine_call(cnt16, disp, oe, ofb):
    mesh = plsc.VectorSubcoreMesh(core_axis_name="c", subcore_axis_name="s")
    fn = functools.partial(
        pl.kernel,
        mesh=mesh,
        out_type=jax.ShapeDtypeStruct((YPAD, H), jnp.float32),
        scratch_types=[
            pltpu.VMEM((16,), jnp.int32),
            pltpu.VMEM((RPW,), jnp.int32),
            pltpu.VMEM((RPW, H), jnp.float32),
            pltpu.SemaphoreType.DMA,
        ],
        compiler_params=pltpu.CompilerParams(needs_layout_passes=False),
    )(_combine_body)
    return fn(cnt16, disp, oe, ofb)


def kernel(x, router_w, router_b, w1, b1, w2, b2, sw1, sb1, sw2, sb2):
    x_f = x.reshape(N, H)
    logits, gtop, topk, dest, meta, x_bf = _router_call(
        x_f, router_w, router_b.reshape(1, E))
    cnt = meta[2:3, :].astype(jnp.int32)
    cnt16 = jnp.broadcast_to(cnt[0, 0], (16,))
    x_pk = lax.bitcast_convert_type(
        x_bf.reshape(N, H // 2, 2), jnp.int32)
    disp, xd_pk = _dispatch_call(cnt16, dest.reshape(N), x_pk)
    x_disp = lax.bitcast_convert_type(
        xd_pk, jnp.bfloat16).reshape(NSLOT, H)
    oe = _ffn_call(x_disp, w1, b1, w2, b2)
    ofb = _fb_call(cnt, x_disp, sw1, sb1.reshape(1, F), sw2,
                   sb2.reshape(1, H))
    y_pad = _combine_call(cnt16, disp, oe, ofb)
    y = y_pad[:N].reshape(x.shape)
    util = meta[0]
    p_mean = meta[1]
    return y, util, util, p_mean, logits, gtop, topk


# speedup vs baseline: 1.6297x; 1.6297x over previous
import functools
import math

import jax
import jax.numpy as jnp
from jax import lax
from jax.experimental import pallas as pl
from jax.experimental.pallas import tpu as pltpu
from jax.experimental.pallas import tpu_sc as plsc

N = 2048
H = 768
F = 3072
E = 8
CAP = 256
NSLOT = 2 * N
NW = 32
RPW = NSLOT // NW
_SQRT2 = math.sqrt(2.0)


def _router_body(x_ref, rw_ref, rb_ref, logits_ref, gtop_ref, topk_ref,
                 dest_ref, meta_ref):
    x = x_ref[...]
    logits = jnp.dot(x, rw_ref[...], preferred_element_type=jnp.float32)
    logits = logits + rb_ref[...]
    logits_ref[...] = logits
    m = jnp.max(logits, axis=-1, keepdims=True)
    ex = jnp.exp(logits - m)
    g = ex / jnp.sum(ex, axis=-1, keepdims=True)

    lane = lax.broadcasted_iota(jnp.int32, (N, E), 1)
    m1 = jnp.max(g, axis=-1, keepdims=True)
    top1 = jnp.min(jnp.where(g == m1, lane, E), axis=-1, keepdims=True)
    oh1 = lane == top1
    g2 = jnp.where(oh1, -jnp.inf, g)
    m2 = jnp.max(g2, axis=-1, keepdims=True)
    top2 = jnp.min(jnp.where(g2 == m2, lane, E), axis=-1, keepdims=True)
    oh2 = lane == top2

    hard = (oh1 | oh2).astype(jnp.float32)
    gt = g * hard
    gt = gt / (jnp.sum(gt, axis=-1, keepdims=True) + 1e-9)
    gtop_ref[...] = gt
    topk_ref[...] = jnp.concatenate([top1, top2], axis=1)

    col = lax.broadcasted_iota(jnp.int32, (CAP, N), 1)

    def cum_excl(cols):
        outs = []
        for i in range(N // CAP):
            row = lax.broadcasted_iota(jnp.int32, (CAP, N), 0) + (i * CAP)
            lb = (col < row).astype(jnp.float32)
            outs.append(jnp.dot(lb, cols, preferred_element_type=jnp.float32))
        return jnp.concatenate(outs, axis=0)

    prim = oh1.astype(jnp.float32)
    counts = jnp.sum(prim, axis=0, keepdims=True)
    used = jnp.minimum(counts, float(CAP))
    free = float(CAP) - used

    cum_p = cum_excl(prim)
    rank_p = jnp.sum(cum_p * prim, axis=-1, keepdims=True)
    keep = rank_p < float(CAP)
    overflow = jnp.logical_not(keep)

    sec = oh2.astype(jnp.float32) * overflow.astype(jnp.float32)
    cum_s = cum_excl(sec)
    rank_s = jnp.sum(cum_s * oh2.astype(jnp.float32), axis=-1, keepdims=True)
    free_sec = jnp.sum(oh2.astype(jnp.float32) * free, axis=-1, keepdims=True)
    used_sec = jnp.sum(oh2.astype(jnp.float32) * used, axis=-1, keepdims=True)
    take2 = overflow & (rank_s < free_sec)

    fb = (overflow & jnp.logical_not(take2)).astype(jnp.float32)
    cum_fb = cum_excl(fb)

    dest = jnp.where(
        keep, top1 * CAP + rank_p.astype(jnp.int32),
        jnp.where(take2,
                  top2 * CAP + (used_sec + rank_s).astype(jnp.int32),
                  N + cum_fb.astype(jnp.int32)))
    dest_ref[...] = dest

    cnt2 = jnp.sum(sec * take2.astype(jnp.float32), axis=0, keepdims=True)
    util = (used + cnt2) / float(N)
    p_mean = jnp.mean(gt, axis=0, keepdims=True)
    fb_cnt = jnp.sum(fb, axis=0, keepdims=True)
    fb_row = jnp.broadcast_to(fb_cnt, (1, E))
    meta_ref[...] = jnp.concatenate(
        [util, p_mean, fb_row, jnp.zeros((5, E), jnp.float32)], axis=0)


def _router_call(x_f, router_w, router_b):
    return pl.pallas_call(
        _router_body,
        out_shape=[
            jax.ShapeDtypeStruct((N, E), jnp.float32),
            jax.ShapeDtypeStruct((N, E), jnp.float32),
            jax.ShapeDtypeStruct((N, 2), jnp.int32),
            jax.ShapeDtypeStruct((N, 1), jnp.int32),
            jax.ShapeDtypeStruct((8, E), jnp.float32),
        ],
    )(x_f, router_w, router_b)


def _dispatch_body(cnt_hbm, dest_hbm, x_hbm, disp_out, xd_out,
                   cnt_v, dest_v, loc_v, gidx_v, rows_v, sem):
    c = lax.axis_index("c")
    s = lax.axis_index("s")
    wid = s * 2 + c
    base = pl.multiple_of(wid * RPW, RPW)

    pltpu.sync_copy(cnt_hbm, cnt_v)
    fb_cnt = jnp.max(cnt_v[...])
    live = (wid < NW // 2) | (base - N < fb_cnt)

    @pl.when(live)
    def _():
        pltpu.sync_copy(dest_hbm, dest_v)

        def init_body(i, carry):
            loc_v[pl.ds(i * 16, 16)] = jnp.full((16,), N, jnp.int32)
            return carry

        lax.fori_loop(0, (RPW + 16) // 16, init_body, 0)

        lanes = lax.iota(jnp.int32, 16)

        def scat_body(i, carry):
            d = dest_v[pl.ds(i * 16, 16)] - base
            ok = (d >= 0) & (d < RPW)
            d = jnp.where(ok, d, RPW + lanes)
            toks = jnp.where(ok, lanes + i * 16, N)
            plsc.store_scatter(loc_v, [d], toks)
            return carry

        lax.fori_loop(0, N // 16, scat_body, 0)

        pltpu.sync_copy(loc_v.at[pl.ds(0, RPW)], disp_out.at[pl.ds(base, RPW)])

        def san_body(i, carry):
            v = loc_v[pl.ds(i * 16, 16)]
            gidx_v[pl.ds(i * 16, 16)] = jnp.where(v >= N, 0, v)
            return carry

        lax.fori_loop(0, RPW // 16, san_body, 0)

        pltpu.async_copy(x_hbm.at[gidx_v], rows_v, sem).wait()
        pltpu.sync_copy(rows_v, xd_out.at[pl.ds(base, RPW)])


def _dispatch_call(cnt16, dest, x_f):
    mesh = plsc.VectorSubcoreMesh(core_axis_name="c", subcore_axis_name="s")
    fn = functools.partial(
        pl.kernel,
        mesh=mesh,
        out_type=[
            jax.ShapeDtypeStruct((NSLOT,), jnp.int32),
            jax.ShapeDtypeStruct((NSLOT, H), jnp.float32),
        ],
        scratch_types=[
            pltpu.VMEM((16,), jnp.int32),
            pltpu.VMEM((N,), jnp.int32),
            pltpu.VMEM((RPW + 16,), jnp.int32),
            pltpu.VMEM((RPW,), jnp.int32),
            pltpu.VMEM((RPW, H), jnp.float32),
            pltpu.SemaphoreType.DMA,
        ],
        compiler_params=pltpu.CompilerParams(needs_layout_passes=False),
    )(_dispatch_body)
    return fn(cnt16, dest, x_f)


def _gelu(h):
    return 0.5 * h * (1.0 + lax.erf(h / _SQRT2))


def _ffn_body(x_ref, w1_ref, b1_ref, w2_ref, b2_ref, o_ref):
    h = jnp.dot(x_ref[...], w1_ref[0], preferred_element_type=jnp.float32)
    h = _gelu(h + b1_ref[0])
    o = jnp.dot(h, w2_ref[0], preferred_element_type=jnp.float32)
    o_ref[...] = o + b2_ref[0]


def _ffn_call(x_disp, w1, b1, w2, b2):
    return pl.pallas_call(
        _ffn_body,
        grid=(E,),
        in_specs=[
            pl.BlockSpec((CAP, H), lambda i: (i, 0)),
            pl.BlockSpec((1, H, F), lambda i: (i, 0, 0)),
            pl.BlockSpec((1, 1, F), lambda i: (i, 0, 0)),
            pl.BlockSpec((1, F, H), lambda i: (i, 0, 0)),
            pl.BlockSpec((1, 1, H), lambda i: (i, 0, 0)),
        ],
        out_specs=pl.BlockSpec((CAP, H), lambda i: (i, 0)),
        out_shape=jax.ShapeDtypeStruct((N, H), jnp.float32),
    )(x_disp, w1.reshape(E, H, F), b1.reshape(E, 1, F),
      w2.reshape(E, F, H), b2.reshape(E, 1, H))


def _fb_body(cnt_ref, x_ref, w1_ref, b1_ref, w2_ref, b2_ref, o_ref):
    i = pl.program_id(0)
    live = i * CAP < cnt_ref[0, 0]

    @pl.when(live)
    def _():
        h = jnp.dot(x_ref[...], w1_ref[...], preferred_element_type=jnp.float32)
        h = _gelu(h + b1_ref[...])
        o = jnp.dot(h, w2_ref[...], preferred_element_type=jnp.float32)
        o_ref[...] = o + b2_ref[...]

    @pl.when(jnp.logical_not(live))
    def _():
        o_ref[...] = jnp.zeros_like(o_ref)


def _fb_call(cnt, x_disp, sw1, sb1, sw2, sb2):
    return pl.pallas_call(
        _fb_body,
        grid=(E,),
        in_specs=[
            pl.BlockSpec(memory_space=pltpu.SMEM),
            pl.BlockSpec((CAP, H), lambda i: (i + E, 0)),
            pl.BlockSpec((H, F), lambda i: (0, 0)),
            pl.BlockSpec((1, F), lambda i: (0, 0)),
            pl.BlockSpec((F, H), lambda i: (0, 0)),
            pl.BlockSpec((1, H), lambda i: (0, 0)),
        ],
        out_specs=pl.BlockSpec((CAP, H), lambda i: (i, 0)),
        out_shape=jax.ShapeDtypeStruct((N, H), jnp.float32),
    )(cnt, x_disp, sw1, sb1, sw2, sb2)


YPAD = N + 8


def _combine_body(cnt_hbm, disp_hbm, oe_hbm, ofb_hbm, y_hbm,
                  cnt_v, idx_v, rows_v, sem):
    c = lax.axis_index("c")
    s = lax.axis_index("s")
    wid = s * 2 + c
    base = pl.multiple_of(wid * RPW, RPW)

    pltpu.sync_copy(cnt_hbm, cnt_v)
    fb_cnt = jnp.max(cnt_v[...])
    live = (wid < NW // 2) | (base - N < fb_cnt)

    @pl.when(live)
    def _():
        pltpu.sync_copy(disp_hbm.at[pl.ds(base, RPW)], idx_v)

        @pl.when(wid < NW // 2)
        def _():
            pltpu.sync_copy(oe_hbm.at[pl.ds(base, RPW)], rows_v)

        @pl.when(wid >= NW // 2)
        def _():
            fb_base = pl.multiple_of(jnp.maximum(base - N, 0), RPW)
            pltpu.sync_copy(ofb_hbm.at[pl.ds(fb_base, RPW)], rows_v)

        pltpu.async_copy(rows_v, y_hbm.at[idx_v], sem).wait()


def _combine_call(cnt16, disp, oe, ofb):
    mesh = plsc.VectorSubcoreMesh(core_axis_name="c", subcore_axis_name="s")
    fn = functools.partial(
        pl.kernel,
        mesh=mesh,
        out_type=jax.ShapeDtypeStruct((YPAD, H), jnp.float32),
        scratch_types=[
            pltpu.VMEM((16,), jnp.int32),
            pltpu.VMEM((RPW,), jnp.int32),
            pltpu.VMEM((RPW, H), jnp.float32),
            pltpu.SemaphoreType.DMA,
        ],
        compiler_params=pltpu.CompilerParams(needs_layout_passes=False),
    )(_combine_body)
    return fn(cnt16, disp, oe, ofb)


def kernel(x, router_w, router_b, w1, b1, w2, b2, sw1, sb1, sw2, sb2):
    x_f = x.reshape(N, H)
    logits, gtop, topk, dest, meta = _router_call(
        x_f, router_w, router_b.reshape(1, E))
    cnt = meta[2:3, :].astype(jnp.int32)
    cnt16 = jnp.broadcast_to(cnt[0, 0], (16,))
    disp, x_disp = _dispatch_call(cnt16, dest.reshape(N), x_f)
    oe = _ffn_call(x_disp, w1, b1, w2, b2)
    ofb = _fb_call(cnt, x_disp, sw1, sb1.reshape(1, F), sw2,
                   sb2.reshape(1, H))
    y_pad = _combine_call(cnt16, disp, oe, ofb)
    y = y_pad[:N].reshape(x.shape)
    util = meta[0]
    p_mean = meta[1]
    return y, util, util, p_mean, logits, gtop, topk
